# baseline (speedup 1.0000x reference)
import functools

import jax
import jax.numpy as jnp
from jax import lax
from jax.experimental import pallas as pl
from jax.experimental.pallas import tpu as pltpu
from jax.experimental.pallas import tpu_sc as plsc

B = 100
N_VAR = 10000
N_CLAUSE = 42500
NCHUNK = 10
C = N_CLAUSE // NCHUNK
C3 = C * 3
GF = C // 16
REM = C - GF * 16
ZETA = 0.001


def _sc_call(v, wgr, pk):
    info = plsc.get_sparse_core_info()
    nc, ns = info.num_cores, info.num_subcores
    nw = nc * ns

    mesh = plsc.VectorSubcoreMesh(core_axis_name="c", subcore_axis_name="s")

    @functools.partial(
        pl.kernel,
        mesh=mesh,
        compiler_params=pltpu.CompilerParams(needs_layout_passes=False),
        out_type=jax.ShapeDtypeStruct((B, N_VAR), jnp.float32),
        scratch_types=[
            pltpu.VMEM((N_VAR,), jnp.float32),
            pltpu.VMEM((N_VAR,), jnp.float32),
            pltpu.VMEM((C3,), jnp.int32),
            pltpu.VMEM((C3,), jnp.int32),
            pltpu.VMEM((C,), jnp.int32),
            pltpu.VMEM((C,), jnp.int32),
            pltpu.SemaphoreType.DMA,
            pltpu.SemaphoreType.DMA,
            pltpu.SemaphoreType.DMA,
        ],
    )
    def k(v_hbm, w_hbm, pk_hbm, out_hbm,
          vrow, acc, pk0, pk1, w0, w1,
          sem0, sem1, semv):
        wid = lax.axis_index("s") * nc + lax.axis_index("c")
        iota = lax.iota(jnp.int32, 16)
        tail_mask = iota < REM
        himask = jnp.full((16,), -65536, jnp.int32)
        idxmask = jnp.full((16,), 0x7FFFFFFF, jnp.int32)
        sgnmask = jnp.full((16,), -2147483648, jnp.int32)

        bufs = ((pk0, w0), (pk1, w1))
        sems = (sem0, sem1)

        def issue(b, c, s):
            p_r, w_r = bufs[s]
            pltpu.async_copy(pk_hbm.at[b, c], p_r, sems[s])
            pltpu.async_copy(w_hbm.at[b, c], w_r, sems[s])

        def wait_chunk(b, s):
            p_r, w_r = bufs[s]
            pltpu.make_async_copy(pk_hbm.at[b, 0], p_r, sems[s]).wait()
            pltpu.make_async_copy(w_hbm.at[b, 0], w_r, sems[s]).wait()

        def flipsign(x, s):
            return plsc.bitcast(plsc.bitcast(x, jnp.int32) ^ s, jnp.float32)

        def process_group(s, rows, mask):
            p_r, w_r = bufs[s]
            r3 = rows * 3
            p0 = plsc.load_gather(p_r, [r3])
            p1 = plsc.load_gather(p_r, [r3 + 1])
            p2 = plsc.load_gather(p_r, [r3 + 2])
            i0 = p0 & idxmask
            i1 = p1 & idxmask
            i2 = p2 & idxmask
            s0 = p0 & sgnmask
            s1 = p1 & sgnmask
            s2 = p2 & sgnmask
            vg0 = plsc.load_gather(vrow, [i0])
            vg1 = plsc.load_gather(vrow, [i1])
            vg2 = plsc.load_gather(vrow, [i2])
            l0 = 1.0 - flipsign(vg0, s0)
            l1 = 1.0 - flipsign(vg1, s1)
            l2 = 1.0 - flipsign(vg2, s2)
            a01 = jnp.minimum(l0, l1)
            a02 = jnp.minimum(l0, l2)
            a12 = jnp.minimum(l1, l2)
            thr = jnp.minimum(a01, l2) + 1e-12
            w = plsc.load_gather(w_r, [rows])
            wg2 = plsc.bitcast(w & himask, jnp.float32)
            wr2 = plsc.bitcast(lax.shift_left(w, 16), jnp.float32)
            z = jnp.zeros((16,), jnp.float32)
            c0 = flipsign(wg2 * a12 + jnp.where(l0 <= thr, wr2 * l0, z), s0)
            c1 = flipsign(wg2 * a02 + jnp.where(l1 <= thr, wr2 * l1, z), s1)
            c2 = flipsign(wg2 * a01 + jnp.where(l2 <= thr, wr2 * l2, z), s2)
            plsc.addupdate_scatter(acc, [i0], c0, mask=mask)
            plsc.addupdate_scatter(acc, [i1], c1, mask=mask)
            plsc.addupdate_scatter(acc, [i2], c2, mask=mask)

        def process_chunk(s):
            @plsc.parallel_loop(0, GF, unroll=8)
            def grp(g):
                process_group(s, g * 16 + iota, None)
            rows = jnp.minimum(GF * 16 + iota, C - 1)
            process_group(s, rows, tail_mask)

        def process_batch(b):
            pltpu.async_copy(v_hbm.at[b], vrow, semv)
            issue(b, 0, 0)

            @plsc.parallel_loop(0, N_VAR // 16, unroll=8)
            def zero_body(i):
                acc[pl.ds(i * 16, 16)] = jnp.zeros((16,), jnp.float32)
            pltpu.make_async_copy(v_hbm.at[b], vrow, semv).wait()

            def chunk_pair(j, carry):
                for s in (0, 1):
                    c = 2 * j + s

                    @pl.when(c + 1 < NCHUNK)
                    def _():
                        issue(b, c + 1, 1 - s)
                    wait_chunk(b, s)
                    process_chunk(s)
                return carry
            lax.fori_loop(0, NCHUNK // 2, chunk_pair, 0)

            @plsc.parallel_loop(0, N_VAR // 16, unroll=8,
                                carry=jnp.zeros((16,), jnp.float32))
            def max_body(i, mx):
                return jnp.maximum(mx, jnp.abs(acc[pl.ds(i * 16, 16)]))
            mx = max_body
            m = jnp.zeros((16,), jnp.float32) + (jnp.max(mx) + 1e-06)
            mi = plsc.bitcast(m, jnp.int32)
            seed = jnp.full((16,), 0x7EF311C3, jnp.int32)
            r = plsc.bitcast(seed - mi, jnp.float32)
            r = r * (2.0 - m * r)
            r = r * (2.0 - m * r)
            r = r * (2.0 - m * r)
            dt = jnp.clip(r, 1e-05, 0.1)

            @plsc.parallel_loop(0, N_VAR // 16, unroll=8)
            def scale_body(i):
                sl = pl.ds(i * 16, 16)
                acc[sl] = acc[sl] * dt
            pltpu.sync_copy(acc, out_hbm.at[b])

        def batch_loop(t, carry):
            b = wid + nw * t

            @pl.when(b < B)
            def _():
                process_batch(b)
            return carry
        lax.fori_loop(0, (B + nw - 1) // nw, batch_loop, 0)

    return k(v, wgr, pk)


def kernel(v, xl, xs, clause_idx, clause_sign):
    neg = ((1 - clause_sign) // 2).astype(jnp.int32)
    pk = (clause_idx | (neg << 31)).reshape(B, NCHUNK, C3)
    wg = (0.5 * (xl * xs)).astype(jnp.bfloat16)
    wr = (0.5 * ((1.0 + ZETA * xl) * (1.0 - xs))).astype(jnp.bfloat16)
    wgu = lax.bitcast_convert_type(wg, jnp.uint16).astype(jnp.uint32)
    wru = lax.bitcast_convert_type(wr, jnp.uint16).astype(jnp.uint32)
    wgr = lax.bitcast_convert_type((wgu << 16) | wru, jnp.int32)
    return _sc_call(v, wgr.reshape(B, NCHUNK, C), pk)


# speedup vs baseline: 1.3986x; 1.3986x over previous
import functools

import jax
import jax.numpy as jnp
from jax import lax
from jax.experimental import pallas as pl
from jax.experimental.pallas import tpu as pltpu
from jax.experimental.pallas import tpu_sc as plsc

B = 100
N_VAR = 10000
N_CLAUSE = 42500
NCHUNK = 10
C = N_CLAUSE // NCHUNK
C3 = C * 3
GF = C // 16
REM = C - GF * 16
ZETA = 0.001


def _sc_call(v, wgr, pk):
    info = plsc.get_sparse_core_info()
    nc, ns = info.num_cores, info.num_subcores
    nw = nc * ns

    mesh = plsc.VectorSubcoreMesh(core_axis_name="c", subcore_axis_name="s")

    @functools.partial(
        pl.kernel,
        mesh=mesh,
        compiler_params=pltpu.CompilerParams(needs_layout_passes=False),
        out_type=jax.ShapeDtypeStruct((B, N_VAR), jnp.float32),
        scratch_types=[
            pltpu.VMEM((N_VAR,), jnp.float32),
            pltpu.VMEM((N_VAR,), jnp.float32),
            pltpu.VMEM((C3,), jnp.int32),
            pltpu.VMEM((C3,), jnp.int32),
            pltpu.VMEM((C,), jnp.int32),
            pltpu.VMEM((C,), jnp.int32),
            pltpu.SemaphoreType.DMA,
            pltpu.SemaphoreType.DMA,
            pltpu.SemaphoreType.DMA,
        ],
    )
    def k(v_hbm, w_hbm, pk_hbm, out_hbm,
          vrow, acc, pk0, pk1, w0, w1,
          sem0, sem1, semv):
        wid = lax.axis_index("s") * nc + lax.axis_index("c")
        iota = lax.iota(jnp.int32, 16)
        tail_mask = iota < REM
        himask = jnp.full((16,), -65536, jnp.int32)
        idxmask = jnp.full((16,), 0x7FFFFFFF, jnp.int32)
        sgnmask = jnp.full((16,), -2147483648, jnp.int32)

        bufs = ((pk0, w0), (pk1, w1))
        sems = (sem0, sem1)

        def issue(b, c, s):
            p_r, w_r = bufs[s]
            pltpu.async_copy(pk_hbm.at[b, c], p_r, sems[s])
            pltpu.async_copy(w_hbm.at[b, c], w_r, sems[s])

        def wait_chunk(b, s):
            p_r, w_r = bufs[s]
            pltpu.make_async_copy(pk_hbm.at[b, 0], p_r, sems[s]).wait()
            pltpu.make_async_copy(w_hbm.at[b, 0], w_r, sems[s]).wait()

        def flipsign(x, s):
            return plsc.bitcast(plsc.bitcast(x, jnp.int32) ^ s, jnp.float32)

        def process_group(s, rows, mask):
            p_r, w_r = bufs[s]
            r3 = rows * 3
            p0 = plsc.load_gather(p_r, [r3])
            p1 = plsc.load_gather(p_r, [r3 + 1])
            p2 = plsc.load_gather(p_r, [r3 + 2])
            i0 = p0 & idxmask
            i1 = p1 & idxmask
            i2 = p2 & idxmask
            s0 = p0 & sgnmask
            s1 = p1 & sgnmask
            s2 = p2 & sgnmask
            vg0 = plsc.load_gather(vrow, [i0])
            vg1 = plsc.load_gather(vrow, [i1])
            vg2 = plsc.load_gather(vrow, [i2])
            l0 = 1.0 - flipsign(vg0, s0)
            l1 = 1.0 - flipsign(vg1, s1)
            l2 = 1.0 - flipsign(vg2, s2)
            a01 = jnp.minimum(l0, l1)
            a02 = jnp.minimum(l0, l2)
            a12 = jnp.minimum(l1, l2)
            thr = jnp.minimum(a01, l2) + 1e-12
            w = plsc.load_gather(w_r, [rows])
            wg2 = plsc.bitcast(w & himask, jnp.float32)
            wr2 = plsc.bitcast(lax.shift_left(w, 16), jnp.float32)
            z = jnp.zeros((16,), jnp.float32)
            c0 = flipsign(wg2 * a12 + jnp.where(l0 <= thr, wr2 * l0, z), s0)
            c1 = flipsign(wg2 * a02 + jnp.where(l1 <= thr, wr2 * l1, z), s1)
            c2 = flipsign(wg2 * a01 + jnp.where(l2 <= thr, wr2 * l2, z), s2)
            plsc.addupdate_scatter(acc, [i0], c0, mask=mask)
            plsc.addupdate_scatter(acc, [i1], c1, mask=mask)
            plsc.addupdate_scatter(acc, [i2], c2, mask=mask)

        def process_chunk(s):
            @plsc.parallel_loop(0, GF, unroll=2)
            def grp(g):
                process_group(s, g * 16 + iota, None)
            rows = jnp.minimum(GF * 16 + iota, C - 1)
            process_group(s, rows, tail_mask)

        def process_batch(b):
            pltpu.async_copy(v_hbm.at[b], vrow, semv)
            issue(b, 0, 0)

            @plsc.parallel_loop(0, N_VAR // 16, unroll=8)
            def zero_body(i):
                acc[pl.ds(i * 16, 16)] = jnp.zeros((16,), jnp.float32)
            pltpu.make_async_copy(v_hbm.at[b], vrow, semv).wait()

            def chunk_pair(j, carry):
                for s in (0, 1):
                    c = 2 * j + s

                    @pl.when(c + 1 < NCHUNK)
                    def _():
                        issue(b, c + 1, 1 - s)
                    wait_chunk(b, s)
                    process_chunk(s)
                return carry
            lax.fori_loop(0, NCHUNK // 2, chunk_pair, 0)

            @plsc.parallel_loop(0, N_VAR // 16, unroll=8,
                                carry=jnp.zeros((16,), jnp.float32))
            def max_body(i, mx):
                return jnp.maximum(mx, jnp.abs(acc[pl.ds(i * 16, 16)]))
            mx = max_body
            m = jnp.zeros((16,), jnp.float32) + (jnp.max(mx) + 1e-06)
            mi = plsc.bitcast(m, jnp.int32)
            seed = jnp.full((16,), 0x7EF311C3, jnp.int32)
            r = plsc.bitcast(seed - mi, jnp.float32)
            r = r * (2.0 - m * r)
            r = r * (2.0 - m * r)
            r = r * (2.0 - m * r)
            dt = jnp.clip(r, 1e-05, 0.1)

            @plsc.parallel_loop(0, N_VAR // 16, unroll=8)
            def scale_body(i):
                sl = pl.ds(i * 16, 16)
                acc[sl] = acc[sl] * dt
            pltpu.sync_copy(acc, out_hbm.at[b])

        def batch_loop(t, carry):
            b = wid + nw * t

            @pl.when(b < B)
            def _():
                process_batch(b)
            return carry
        lax.fori_loop(0, (B + nw - 1) // nw, batch_loop, 0)

    return k(v, wgr, pk)


def kernel(v, xl, xs, clause_idx, clause_sign):
    neg = ((1 - clause_sign) // 2).astype(jnp.int32)
    pk = (clause_idx | (neg << 31)).reshape(B, NCHUNK, C3)
    wg = (0.5 * (xl * xs)).astype(jnp.bfloat16)
    wr = (0.5 * ((1.0 + ZETA * xl) * (1.0 - xs))).astype(jnp.bfloat16)
    wgu = lax.bitcast_convert_type(wg, jnp.uint16).astype(jnp.uint32)
    wru = lax.bitcast_convert_type(wr, jnp.uint16).astype(jnp.uint32)
    wgr = lax.bitcast_convert_type((wgu << 16) | wru, jnp.int32)
    return _sc_call(v, wgr.reshape(B, NCHUNK, C), pk)


# speedup vs baseline: 1.4015x; 1.0020x over previous
import functools

import jax
import jax.numpy as jnp
from jax import lax
from jax.experimental import pallas as pl
from jax.experimental.pallas import tpu as pltpu
from jax.experimental.pallas import tpu_sc as plsc

B = 100
N_VAR = 10000
N_CLAUSE = 42500
NCHUNK = 10
C = N_CLAUSE // NCHUNK
C3 = C * 3
GF = C // 16
REM = C - GF * 16
ZETA = 0.001


def _sc_call(v, wgr, pk):
    info = plsc.get_sparse_core_info()
    nc, ns = info.num_cores, info.num_subcores
    nw = nc * ns

    mesh = plsc.VectorSubcoreMesh(core_axis_name="c", subcore_axis_name="s")

    @functools.partial(
        pl.kernel,
        mesh=mesh,
        compiler_params=pltpu.CompilerParams(needs_layout_passes=False),
        out_type=jax.ShapeDtypeStruct((B, N_VAR), jnp.float32),
        scratch_types=[
            pltpu.VMEM((N_VAR,), jnp.float32),
            pltpu.VMEM((N_VAR,), jnp.float32),
            pltpu.VMEM((C3,), jnp.int32),
            pltpu.VMEM((C3,), jnp.int32),
            pltpu.VMEM((C,), jnp.int32),
            pltpu.VMEM((C,), jnp.int32),
            pltpu.SemaphoreType.DMA,
            pltpu.SemaphoreType.DMA,
            pltpu.SemaphoreType.DMA,
        ],
    )
    def k(v_hbm, w_hbm, pk_hbm, out_hbm,
          vrow, acc, pk0, pk1, w0, w1,
          sem0, sem1, semv):
        wid = lax.axis_index("s") * nc + lax.axis_index("c")
        iota = lax.iota(jnp.int32, 16)
        tail_mask = iota < REM
        himask = jnp.full((16,), -65536, jnp.int32)
        idxmask = jnp.full((16,), 0x7FFFFFFF, jnp.int32)
        sgnmask = jnp.full((16,), -2147483648, jnp.int32)

        bufs = ((pk0, w0), (pk1, w1))
        sems = (sem0, sem1)

        def issue(b, c, s):
            p_r, w_r = bufs[s]
            pltpu.async_copy(pk_hbm.at[b, c], p_r, sems[s])
            pltpu.async_copy(w_hbm.at[b, c], w_r, sems[s])

        def wait_chunk(b, s):
            p_r, w_r = bufs[s]
            pltpu.make_async_copy(pk_hbm.at[b, 0], p_r, sems[s]).wait()
            pltpu.make_async_copy(w_hbm.at[b, 0], w_r, sems[s]).wait()

        def flipsign(x, s):
            return plsc.bitcast(plsc.bitcast(x, jnp.int32) ^ s, jnp.float32)

        def process_group(s, rows, mask):
            p_r, w_r = bufs[s]
            r3 = rows * 3
            p0 = plsc.load_gather(p_r, [r3])
            p1 = plsc.load_gather(p_r, [r3 + 1])
            p2 = plsc.load_gather(p_r, [r3 + 2])
            i0 = p0 & idxmask
            i1 = p1 & idxmask
            i2 = p2 & idxmask
            s0 = p0 & sgnmask
            s1 = p1 & sgnmask
            s2 = p2 & sgnmask
            vg0 = plsc.load_gather(vrow, [i0])
            vg1 = plsc.load_gather(vrow, [i1])
            vg2 = plsc.load_gather(vrow, [i2])
            l0 = 1.0 - flipsign(vg0, s0)
            l1 = 1.0 - flipsign(vg1, s1)
            l2 = 1.0 - flipsign(vg2, s2)
            a01 = jnp.minimum(l0, l1)
            a02 = jnp.minimum(l0, l2)
            a12 = jnp.minimum(l1, l2)
            thr = jnp.minimum(a01, l2) + 1e-12
            w = plsc.load_gather(w_r, [rows])
            wg2 = plsc.bitcast(w & himask, jnp.float32)
            wr2 = plsc.bitcast(lax.shift_left(w, 16), jnp.float32)
            z = jnp.zeros((16,), jnp.float32)
            c0 = flipsign(wg2 * a12 + jnp.where(l0 <= thr, wr2 * l0, z), s0)
            c1 = flipsign(wg2 * a02 + jnp.where(l1 <= thr, wr2 * l1, z), s1)
            c2 = flipsign(wg2 * a01 + jnp.where(l2 <= thr, wr2 * l2, z), s2)
            plsc.addupdate_scatter(acc, [i0], c0, mask=mask)
            plsc.addupdate_scatter(acc, [i1], c1, mask=mask)
            plsc.addupdate_scatter(acc, [i2], c2, mask=mask)

        def process_chunk(s):
            @plsc.parallel_loop(0, GF, unroll=1)
            def grp(g):
                process_group(s, g * 16 + iota, None)
            rows = jnp.minimum(GF * 16 + iota, C - 1)
            process_group(s, rows, tail_mask)

        def process_batch(b):
            pltpu.async_copy(v_hbm.at[b], vrow, semv)
            issue(b, 0, 0)

            @plsc.parallel_loop(0, N_VAR // 16, unroll=8)
            def zero_body(i):
                acc[pl.ds(i * 16, 16)] = jnp.zeros((16,), jnp.float32)
            pltpu.make_async_copy(v_hbm.at[b], vrow, semv).wait()

            def chunk_pair(j, carry):
                for s in (0, 1):
                    c = 2 * j + s

                    @pl.when(c + 1 < NCHUNK)
                    def _():
                        issue(b, c + 1, 1 - s)
                    wait_chunk(b, s)
                    process_chunk(s)
                return carry
            lax.fori_loop(0, NCHUNK // 2, chunk_pair, 0)

            @plsc.parallel_loop(0, N_VAR // 16, unroll=8,
                                carry=jnp.zeros((16,), jnp.float32))
            def max_body(i, mx):
                return jnp.maximum(mx, jnp.abs(acc[pl.ds(i * 16, 16)]))
            mx = max_body
            m = jnp.zeros((16,), jnp.float32) + (jnp.max(mx) + 1e-06)
            mi = plsc.bitcast(m, jnp.int32)
            seed = jnp.full((16,), 0x7EF311C3, jnp.int32)
            r = plsc.bitcast(seed - mi, jnp.float32)
            r = r * (2.0 - m * r)
            r = r * (2.0 - m * r)
            r = r * (2.0 - m * r)
            dt = jnp.clip(r, 1e-05, 0.1)

            @plsc.parallel_loop(0, N_VAR // 16, unroll=8)
            def scale_body(i):
                sl = pl.ds(i * 16, 16)
                acc[sl] = acc[sl] * dt
            pltpu.sync_copy(acc, out_hbm.at[b])

        def batch_loop(t, carry):
            b = wid + nw * t

            @pl.when(b < B)
            def _():
                process_batch(b)
            return carry
        lax.fori_loop(0, (B + nw - 1) // nw, batch_loop, 0)

    return k(v, wgr, pk)


def kernel(v, xl, xs, clause_idx, clause_sign):
    neg = ((1 - clause_sign) // 2).astype(jnp.int32)
    pk = (clause_idx | (neg << 31)).reshape(B, NCHUNK, C3)
    wg = (0.5 * (xl * xs)).astype(jnp.bfloat16)
    wr = (0.5 * ((1.0 + ZETA * xl) * (1.0 - xs))).astype(jnp.bfloat16)
    wgu = lax.bitcast_convert_type(wg, jnp.uint16).astype(jnp.uint32)
    wru = lax.bitcast_convert_type(wr, jnp.uint16).astype(jnp.uint32)
    wgr = lax.bitcast_convert_type((wgu << 16) | wru, jnp.int32)
    return _sc_call(v, wgr.reshape(B, NCHUNK, C), pk)


# speedup vs baseline: 1.4222x; 1.0148x over previous
import functools

import jax
import jax.numpy as jnp
from jax import lax
from jax.experimental import pallas as pl
from jax.experimental.pallas import tpu as pltpu
from jax.experimental.pallas import tpu_sc as plsc

B = 100
N_VAR = 10000
N_CLAUSE = 42500
NCHUNK = 10
C = N_CLAUSE // NCHUNK
C3 = C * 3
GF = C // 16
REM = C - GF * 16
ZETA = 0.001


def _sc_call(v, wgr, pk):
    info = plsc.get_sparse_core_info()
    nc, ns = info.num_cores, info.num_subcores
    nw = nc * ns

    full_rounds = B // nw
    full = full_rounds * nw
    tail = B - full
    use_split = tail > 0 and (tail & (tail - 1)) == 0
    ptail = tail if use_split else 1

    mesh = plsc.VectorSubcoreMesh(core_axis_name="c", subcore_axis_name="s")

    @functools.partial(
        pl.kernel,
        mesh=mesh,
        compiler_params=pltpu.CompilerParams(needs_layout_passes=False),
        out_type=(
            jax.ShapeDtypeStruct((B, N_VAR), jnp.float32),
            jax.ShapeDtypeStruct((ptail, NCHUNK, N_VAR), jnp.float32),
        ),
        scratch_types=[
            pltpu.VMEM((N_VAR,), jnp.float32),
            pltpu.VMEM((N_VAR,), jnp.float32),
            pltpu.VMEM((C3,), jnp.int32),
            pltpu.VMEM((C3,), jnp.int32),
            pltpu.VMEM((C,), jnp.int32),
            pltpu.VMEM((C,), jnp.int32),
            pltpu.SemaphoreType.DMA,
            pltpu.SemaphoreType.DMA,
            pltpu.SemaphoreType.DMA,
        ],
    )
    def k(v_hbm, w_hbm, pk_hbm, out_hbm, part_hbm,
          vrow, acc, pk0, pk1, w0, w1,
          sem0, sem1, semv):
        wid = lax.axis_index("s") * nc + lax.axis_index("c")
        iota = lax.iota(jnp.int32, 16)
        tail_mask = iota < REM
        himask = jnp.full((16,), -65536, jnp.int32)
        idxmask = jnp.full((16,), 0x7FFFFFFF, jnp.int32)
        sgnmask = jnp.full((16,), -2147483648, jnp.int32)

        bufs = ((pk0, w0), (pk1, w1))
        sems = (sem0, sem1)

        def issue(b, c, s):
            p_r, w_r = bufs[s]
            pltpu.async_copy(pk_hbm.at[b, c], p_r, sems[s])
            pltpu.async_copy(w_hbm.at[b, c], w_r, sems[s])

        def wait_chunk(b, s):
            p_r, w_r = bufs[s]
            pltpu.make_async_copy(pk_hbm.at[b, 0], p_r, sems[s]).wait()
            pltpu.make_async_copy(w_hbm.at[b, 0], w_r, sems[s]).wait()

        def flipsign(x, s):
            return plsc.bitcast(plsc.bitcast(x, jnp.int32) ^ s, jnp.float32)

        def process_group(s, rows, mask):
            p_r, w_r = bufs[s]
            r3 = rows * 3
            p0 = plsc.load_gather(p_r, [r3])
            p1 = plsc.load_gather(p_r, [r3 + 1])
            p2 = plsc.load_gather(p_r, [r3 + 2])
            i0 = p0 & idxmask
            i1 = p1 & idxmask
            i2 = p2 & idxmask
            s0 = p0 & sgnmask
            s1 = p1 & sgnmask
            s2 = p2 & sgnmask
            vg0 = plsc.load_gather(vrow, [i0])
            vg1 = plsc.load_gather(vrow, [i1])
            vg2 = plsc.load_gather(vrow, [i2])
            l0 = 1.0 - flipsign(vg0, s0)
            l1 = 1.0 - flipsign(vg1, s1)
            l2 = 1.0 - flipsign(vg2, s2)
            a01 = jnp.minimum(l0, l1)
            a02 = jnp.minimum(l0, l2)
            a12 = jnp.minimum(l1, l2)
            thr = jnp.minimum(a01, l2) + 1e-12
            w = plsc.load_gather(w_r, [rows])
            wg2 = plsc.bitcast(w & himask, jnp.float32)
            wr2 = plsc.bitcast(lax.shift_left(w, 16), jnp.float32)
            z = jnp.zeros((16,), jnp.float32)
            c0 = flipsign(wg2 * a12 + jnp.where(l0 <= thr, wr2 * l0, z), s0)
            c1 = flipsign(wg2 * a02 + jnp.where(l1 <= thr, wr2 * l1, z), s1)
            c2 = flipsign(wg2 * a01 + jnp.where(l2 <= thr, wr2 * l2, z), s2)
            plsc.addupdate_scatter(acc, [i0], c0, mask=mask)
            plsc.addupdate_scatter(acc, [i1], c1, mask=mask)
            plsc.addupdate_scatter(acc, [i2], c2, mask=mask)

        def process_chunk(s):
            @plsc.parallel_loop(0, GF, unroll=1)
            def grp(g):
                process_group(s, g * 16 + iota, None)
            rows = jnp.minimum(GF * 16 + iota, C - 1)
            process_group(s, rows, tail_mask)

        def zero_acc():
            @plsc.parallel_loop(0, N_VAR // 16, unroll=8)
            def zero_body(i):
                acc[pl.ds(i * 16, 16)] = jnp.zeros((16,), jnp.float32)

        def process_batch(b):
            pltpu.async_copy(v_hbm.at[b], vrow, semv)
            issue(b, 0, 0)
            zero_acc()
            pltpu.make_async_copy(v_hbm.at[b], vrow, semv).wait()

            def chunk_pair(j, carry):
                for s in (0, 1):
                    c = 2 * j + s

                    @pl.when(c + 1 < NCHUNK)
                    def _():
                        issue(b, c + 1, 1 - s)
                    wait_chunk(b, s)
                    process_chunk(s)
                return carry
            lax.fori_loop(0, NCHUNK // 2, chunk_pair, 0)

            @plsc.parallel_loop(0, N_VAR // 16, unroll=8,
                                carry=jnp.zeros((16,), jnp.float32))
            def max_body(i, mx):
                return jnp.maximum(mx, jnp.abs(acc[pl.ds(i * 16, 16)]))
            mx = max_body
            m = jnp.zeros((16,), jnp.float32) + (jnp.max(mx) + 1e-06)
            mi = plsc.bitcast(m, jnp.int32)
            seed = jnp.full((16,), 0x7EF311C3, jnp.int32)
            r = plsc.bitcast(seed - mi, jnp.float32)
            r = r * (2.0 - m * r)
            r = r * (2.0 - m * r)
            r = r * (2.0 - m * r)
            dt = jnp.clip(r, 1e-05, 0.1)

            @plsc.parallel_loop(0, N_VAR // 16, unroll=8)
            def scale_body(i):
                sl = pl.ds(i * 16, 16)
                acc[sl] = acc[sl] * dt
            pltpu.sync_copy(acc, out_hbm.at[b])

        if use_split:
            sh = tail.bit_length() - 1
            tailu = tail * NCHUNK
            for r in range(-(-tailu // nw)):
                u = wid + nw * r
                slot = r % 2

                @pl.when(u < tailu)
                def _():
                    tb = u & (tail - 1)
                    c = lax.shift_right_logical(u, sh)
                    b = full + tb
                    pltpu.async_copy(v_hbm.at[b], vrow, semv)
                    issue(b, c, slot)
                    zero_acc()
                    pltpu.make_async_copy(v_hbm.at[b], vrow, semv).wait()
                    wait_chunk(b, slot)
                    process_chunk(slot)
                    pltpu.sync_copy(acc, part_hbm.at[tb, c])

        def batch_loop(t, carry):
            b = wid + nw * t
            if use_split:
                process_batch(b)
            else:
                @pl.when(b < B)
                def _():
                    process_batch(b)
            return carry
        rounds = full_rounds if use_split else (B + nw - 1) // nw
        lax.fori_loop(0, rounds, batch_loop, 0)

    out, part = k(v, wgr, pk)
    if not use_split:
        return out

    def combine_body(p_ref, o_ref):
        g = jnp.sum(p_ref[0], axis=0, keepdims=True)
        m = jnp.max(jnp.abs(g)) + 1e-06
        dt = jnp.clip(1.0 / m, 1e-05, 0.1)
        o_ref[0] = g * dt

    tail_out = pl.pallas_call(
        combine_body,
        grid=(tail,),
        in_specs=[pl.BlockSpec((1, NCHUNK, N_VAR), lambda t: (t, 0, 0))],
        out_specs=pl.BlockSpec((1, 1, N_VAR), lambda t: (t, 0, 0)),
        out_shape=jax.ShapeDtypeStruct((tail, 1, N_VAR), jnp.float32),
    )(part)
    return jnp.concatenate([out[:full], tail_out.reshape(tail, N_VAR)], axis=0)


def kernel(v, xl, xs, clause_idx, clause_sign):
    neg = ((1 - clause_sign) // 2).astype(jnp.int32)
    pk = (clause_idx | (neg << 31)).reshape(B, NCHUNK, C3)
    wg = (0.5 * (xl * xs)).astype(jnp.bfloat16)
    wr = (0.5 * ((1.0 + ZETA * xl) * (1.0 - xs))).astype(jnp.bfloat16)
    wgu = lax.bitcast_convert_type(wg, jnp.uint16).astype(jnp.uint32)
    wru = lax.bitcast_convert_type(wr, jnp.uint16).astype(jnp.uint32)
    wgr = lax.bitcast_convert_type((wgu << 16) | wru, jnp.int32)
    return _sc_call(v, wgr.reshape(B, NCHUNK, C), pk)
